# baseline (speedup 1.0000x reference)
import functools

import jax
import jax.numpy as jnp
from jax import lax
from jax.experimental import pallas as pl
from jax.experimental.pallas import tpu as pltpu
from jax.experimental.pallas import tpu_sc as plsc

_N = 10000
_E = 320000
_D = 128
_H = 128
_P = 100

_NP = 10240
_EP = 327680
_EROWS = 2560
_CHUNK = 128
_TILES = 16
_RPT = _NP // _TILES
_ECHUNKS = _EROWS // _TILES
_BLK = 1024
_NBLK = _NP // _BLK


def _deg_body(dst_hbm, out_hbm, dst_v, ones_v, buf_v, acc_sh):
    c = lax.axis_index("c")
    s = lax.axis_index("s")
    w = c * _TILES + s
    rows = _EROWS // 32
    r0 = s * _RPT

    def fill_zero(i, carry):
        j = i // 8
        k = i % 8
        buf_v[j, pl.ds(k * 16, 16)] = jnp.zeros((16,), jnp.float32)
        return carry
    lax.fori_loop(0, 160 * 8, fill_zero, 0)

    def fill_one(i, carry):
        j = i // 8
        k = i % 8
        ones_v[j, pl.ds(k * 16, 16)] = jnp.ones((16,), jnp.float32)
        return carry
    lax.fori_loop(0, _CHUNK * 8, fill_one, 0)

    def init(t, carry):
        pltpu.sync_copy(buf_v, acc_sh.at[pl.ds(r0 + t * 160, 160)])
        return carry
    lax.fori_loop(0, _RPT // 160, init, 0)

    pltpu.sync_copy(dst_hbm.at[pl.ds(w * rows, rows)], dst_v)
    plsc.subcore_barrier()

    def upd(j, carry):
        pltpu.sync_copy(ones_v, acc_sh.at[dst_v.at[j]], add=True)
        return carry
    lax.fori_loop(0, rows, upd, 0)

    plsc.subcore_barrier()

    def dump(t, carry):
        pltpu.sync_copy(acc_sh.at[pl.ds(r0 + t * 160, 160)], buf_v)
        pltpu.sync_copy(buf_v, out_hbm.at[c, pl.ds(r0 + t * 160, 160)])
        return carry
    lax.fori_loop(0, _RPT // 160, dump, 0)


def _sc_degree(dst2d):
    mesh = plsc.VectorSubcoreMesh(core_axis_name="c", subcore_axis_name="s")
    f = pl.kernel(
        _deg_body,
        mesh=mesh,
        out_type=jax.ShapeDtypeStruct((2, _NP, _H), jnp.float32),
        scratch_types=[
            pltpu.VMEM((_EROWS // 32, _CHUNK), jnp.int32),
            pltpu.VMEM((_CHUNK, _H), jnp.float32),
            pltpu.VMEM((160, _H), jnp.float32),
            pltpu.VMEM_SHARED((_NP, _H), jnp.float32),
        ],
    )
    return f(dst2d)


_HALF = _NP // 2
_ACCR = _HALF + 128
_RPT2 = _HALF // _TILES
_DUMP = 160


def _seg_body(src_hbm, dst_hbm, hs_hbm, out_hbm, src_v, dst_v, rowa_v, rowb_v,
              acc_sh, sema, semb):
    c = lax.axis_index("c")
    s = lax.axis_index("s")
    r0 = s * _RPT2
    buf = rowa_v.at[pl.ds(0, 64)]

    def init(t, carry):
        pltpu.sync_copy(hs_hbm.at[pl.ds(c * _HALF + r0 + t * 64, 64)], buf)
        pltpu.sync_copy(buf, acc_sh.at[pl.ds(r0 + t * 64, 64)])
        return carry
    lax.fori_loop(0, _RPT2 // 64, init, 0)

    pltpu.sync_copy(src_hbm.at[pl.ds((c * _TILES + s) * _ECHUNKS, _ECHUNKS)], src_v)
    pltpu.sync_copy(dst_hbm.at[pl.ds((c * _TILES + s) * _ECHUNKS, _ECHUNKS)], dst_v)
    plsc.subcore_barrier()

    pltpu.async_copy(hs_hbm.at[src_v.at[0]], rowa_v, sema)

    def pair(t, carry):
        j = t * 2
        pltpu.make_async_copy(hs_hbm.at[pl.ds(0, _CHUNK)], rowa_v, sema).wait()
        pltpu.async_copy(hs_hbm.at[src_v.at[j + 1]], rowb_v, semb)
        pltpu.sync_copy(rowa_v, acc_sh.at[dst_v.at[j]], add=True)
        pltpu.make_async_copy(hs_hbm.at[pl.ds(0, _CHUNK)], rowb_v, semb).wait()

        @pl.when(j + 2 < _ECHUNKS)
        def _():
            pltpu.async_copy(hs_hbm.at[src_v.at[j + 2]], rowa_v, sema)

        pltpu.sync_copy(rowb_v, acc_sh.at[dst_v.at[j + 1]], add=True)
        return carry
    lax.fori_loop(0, _ECHUNKS // 2, pair, 0)

    plsc.subcore_barrier()

    def dump(t, carry):
        pltpu.sync_copy(acc_sh.at[pl.ds(r0 + t * 64, 64)], buf)
        pltpu.sync_copy(buf, out_hbm.at[c, pl.ds(r0 + t * 64, 64)])
        return carry
    lax.fori_loop(0, _RPT2 // 64, dump, 0)


def _sc_segsum(src2d, dstc, hs):
    mesh = plsc.VectorSubcoreMesh(core_axis_name="c", subcore_axis_name="s")
    f = pl.kernel(
        _seg_body,
        mesh=mesh,
        out_type=jax.ShapeDtypeStruct((2, _HALF, _H), jnp.float32),
        scratch_types=[
            pltpu.VMEM((_ECHUNKS, _CHUNK), jnp.int32),
            pltpu.VMEM((_ECHUNKS, _CHUNK), jnp.int32),
            pltpu.VMEM((_CHUNK, _H), jnp.float32),
            pltpu.VMEM((_CHUNK, _H), jnp.float32),
            pltpu.VMEM_SHARED((_ACCR, _H), jnp.float32),
            pltpu.SemaphoreType.DMA,
            pltpu.SemaphoreType.DMA,
        ],
    )
    return f(src2d, dstc, hs)


def _mm_scale_body(x_ref, w_ref, degp_ref, out_ref):
    dinv = lax.rsqrt(degp_ref[0, :, 0] + degp_ref[1, :, 0] + 1.0)
    h = jnp.dot(x_ref[...], w_ref[...], preferred_element_type=jnp.float32)
    out_ref[...] = h * dinv[:, None]


def _tc_matmul_scale(x_pad, w, degp):
    return pl.pallas_call(
        _mm_scale_body,
        grid=(_NBLK,),
        in_specs=[
            pl.BlockSpec((_BLK, _D), lambda i: (i, 0)),
            pl.BlockSpec((_D, _H), lambda i: (0, 0)),
            pl.BlockSpec((2, _BLK, _H), lambda i: (0, i, 0)),
        ],
        out_specs=pl.BlockSpec((_BLK, _H), lambda i: (i, 0)),
        out_shape=jax.ShapeDtypeStruct((_NP, _H), jnp.float32),
    )(x_pad, w, degp)


def _combine_body(agg_ref, b_ref, degp_ref, z_ref, st_ref, acc):
    i = pl.program_id(0)
    dinv = lax.rsqrt(degp_ref[0, :, 0] + degp_ref[1, :, 0] + 1.0)
    z = agg_ref[0]
    z = z * dinv[:, None] + b_ref[...]
    z_ref[...] = z
    rows = i * _BLK + lax.broadcasted_iota(jnp.int32, (_BLK, 1), 0)
    zm = jnp.where(rows < _N, z, 0.0)
    part = jnp.concatenate([
        jnp.sum(zm, axis=0, keepdims=True),
        jnp.sum(zm * zm, axis=0, keepdims=True),
    ], axis=0)

    @pl.when(i == 0)
    def _():
        acc[...] = jnp.zeros_like(acc)

    acc[...] += part
    st_ref[...] = acc[...]


def _tc_combine(agg, b_row, degp):
    return pl.pallas_call(
        _combine_body,
        grid=(_NBLK,),
        in_specs=[
            pl.BlockSpec((1, _BLK, _H), lambda i: (i // 5, i % 5, 0)),
            pl.BlockSpec((1, _H), lambda i: (0, 0)),
            pl.BlockSpec((2, _BLK, _H), lambda i: (0, i, 0)),
        ],
        out_specs=[
            pl.BlockSpec((_BLK, _H), lambda i: (i, 0)),
            pl.BlockSpec((2, _H), lambda i: (0, 0)),
        ],
        out_shape=[
            jax.ShapeDtypeStruct((_NP, _H), jnp.float32),
            jax.ShapeDtypeStruct((2, _H), jnp.float32),
        ],
        scratch_shapes=[pltpu.VMEM((2, _H), jnp.float32)],
    )(agg, b_row, degp)


def _norm_mm_body(z_ref, st_ref, w_ref, degp_ref, out_ref):
    mean = st_ref[0, :] / float(_N)
    var = st_ref[1, :] / float(_N) - mean * mean
    inv = lax.rsqrt(var + 1e-5)
    hn = jnp.maximum((z_ref[...] - mean[None, :]) * inv[None, :], 0.0)
    h = jnp.dot(hn, w_ref[...], preferred_element_type=jnp.float32)
    dinv = lax.rsqrt(degp_ref[0, :, 0] + degp_ref[1, :, 0] + 1.0)
    out_ref[...] = h * dinv[:, None]


def _tc_norm_matmul(z, st, w, degp):
    return pl.pallas_call(
        _norm_mm_body,
        grid=(_NBLK,),
        in_specs=[
            pl.BlockSpec((_BLK, _H), lambda i: (i, 0)),
            pl.BlockSpec((2, _H), lambda i: (0, 0)),
            pl.BlockSpec((_H, _H), lambda i: (0, 0)),
            pl.BlockSpec((2, _BLK, _H), lambda i: (0, i, 0)),
        ],
        out_specs=pl.BlockSpec((_BLK, _H), lambda i: (i, 0)),
        out_shape=jax.ShapeDtypeStruct((_NP, _H), jnp.float32),
    )(z, st, w, degp)


def _final_body(cur_ref, z_ref, zc_ref, st_ref, part_ref, h_ref, p_ref, acc):
    i = pl.program_id(0)
    mean = st_ref[0, :] / float(_N)
    var = st_ref[1, :] / float(_N) - mean * mean
    inv = lax.rsqrt(var + 1e-5)
    hb = jnp.maximum((z_ref[...] - mean[None, :]) * inv[None, :], 0.0)
    h_ref[...] = hb
    sub = cur_ref[0] % 8
    lane = lax.broadcasted_iota(jnp.int32, (8, 1), 0)
    zc = jnp.sum(jnp.where(lane == sub, zc_ref[...], 0.0), axis=0, keepdims=True)
    hc = jnp.maximum((zc - mean[None, :]) * inv[None, :], 0.0)
    scores = jnp.sum(hb * hc, axis=1, keepdims=True)
    rows = i * _BLK + lax.broadcasted_iota(jnp.int32, (_BLK, 1), 0)
    scores = jnp.where(rows < _N, scores, 0.0)
    contrib = jnp.sum(part_ref[...] * scores, axis=0, keepdims=True)

    @pl.when(i == 0)
    def _():
        acc[...] = jnp.zeros_like(acc)

    acc[...] += contrib
    p_ref[...] = acc[...]


def _tc_final(cur, z, st, part_pad):
    grid_spec = pltpu.PrefetchScalarGridSpec(
        num_scalar_prefetch=1,
        grid=(_NBLK,),
        in_specs=[
            pl.BlockSpec((_BLK, _H), lambda i, cur: (i, 0)),
            pl.BlockSpec((8, _H), lambda i, cur: (cur[0] // 8, 0)),
            pl.BlockSpec((2, _H), lambda i, cur: (0, 0)),
            pl.BlockSpec((_BLK, _H), lambda i, cur: (i, 0)),
        ],
        out_specs=[
            pl.BlockSpec((_BLK, _H), lambda i, cur: (i, 0)),
            pl.BlockSpec((1, _H), lambda i, cur: (0, 0)),
        ],
        scratch_shapes=[pltpu.VMEM((1, _H), jnp.float32)],
    )
    return pl.pallas_call(
        _final_body,
        grid_spec=grid_spec,
        out_shape=[
            jax.ShapeDtypeStruct((_N, _H), jnp.float32),
            jax.ShapeDtypeStruct((1, _H), jnp.float32),
        ],
    )(cur, z, z, st, part_pad)


def kernel(x, edge_index, curr_node_id, partitions, node_weights, W1, b1, W2, b2):
    del node_weights
    e = edge_index.shape[1]
    pad = jnp.full((_EP - e,), _N, dtype=jnp.int32)
    src_p = jnp.concatenate([edge_index[0].astype(jnp.int32), pad])
    dst_p = jnp.concatenate([edge_index[1].astype(jnp.int32), pad])
    src2d = src_p.reshape(_EROWS, _CHUNK)
    dst2d = dst_p.reshape(_EROWS, _CHUNK)
    sent = _HALF + (jnp.arange(_EP, dtype=jnp.int32) % 128)
    is0 = dst_p < _HALF
    dst_c0 = jnp.where(is0, dst_p, sent)
    dst_c1 = jnp.where(is0, sent, dst_p - _HALF)
    dstc = jnp.concatenate([dst_c0, dst_c1]).reshape(2 * _EROWS, _CHUNK)
    src_c0 = jnp.where(is0, src_p, _N)
    src_c1 = jnp.where(is0, _N, src_p)
    srcc = jnp.concatenate([src_c0, src_c1]).reshape(2 * _EROWS, _CHUNK)
    x_pad = jnp.pad(x, ((0, _NP - _N), (0, 0)))
    part_pad = jnp.pad(partitions, ((0, _NP - _N), (0, _H - _P)))

    degp = _sc_degree(dst2d)
    hs1 = _tc_matmul_scale(x_pad, W1, degp)
    agg1 = _sc_segsum(srcc, dstc, hs1)
    z1, st1 = _tc_combine(agg1, b1.reshape(1, _H), degp)
    hs2 = _tc_norm_matmul(z1, st1, W2, degp)
    agg2 = _sc_segsum(srcc, dstc, hs2)
    z2, st2 = _tc_combine(agg2, b2.reshape(1, _H), degp)

    cur = jnp.asarray(curr_node_id, dtype=jnp.int32).reshape(1)
    h, p = _tc_final(cur, z2, st2, part_pad)
    return p[:, :_P], h


# speedup vs baseline: 15.7161x; 15.7161x over previous
import functools

import jax
import jax.numpy as jnp
from jax import lax
from jax.experimental import pallas as pl
from jax.experimental.pallas import tpu as pltpu
from jax.experimental.pallas import tpu_sc as plsc

_N = 10000
_E = 320000
_D = 128
_H = 128
_P = 100

_NP = 10240
_EP = 327680
_EROWS = 2560
_CHUNK = 128
_TILES = 16
_RPT = _NP // _TILES
_ECHUNKS = _EROWS // _TILES
_BLK = 1024
_NBLK = _NP // _BLK


def _deg_body(dst_hbm, out_hbm, dst_v, ones_v, buf_v, acc_sh):
    c = lax.axis_index("c")
    s = lax.axis_index("s")
    w = c * _TILES + s
    rows = _EROWS // 32
    r0 = s * _RPT

    def fill_zero(i, carry):
        j = i // 8
        k = i % 8
        buf_v[j, pl.ds(k * 16, 16)] = jnp.zeros((16,), jnp.float32)
        return carry
    lax.fori_loop(0, 160 * 8, fill_zero, 0)

    def fill_one(i, carry):
        j = i // 8
        k = i % 8
        ones_v[j, pl.ds(k * 16, 16)] = jnp.ones((16,), jnp.float32)
        return carry
    lax.fori_loop(0, _CHUNK * 8, fill_one, 0)

    def init(t, carry):
        pltpu.sync_copy(buf_v, acc_sh.at[pl.ds(r0 + t * 160, 160)])
        return carry
    lax.fori_loop(0, _RPT // 160, init, 0)

    pltpu.sync_copy(dst_hbm.at[pl.ds(w * rows, rows)], dst_v)
    plsc.subcore_barrier()

    def upd(j, carry):
        pltpu.sync_copy(ones_v, acc_sh.at[dst_v.at[j]], add=True)
        return carry
    lax.fori_loop(0, rows, upd, 0)

    plsc.subcore_barrier()

    def dump(t, carry):
        pltpu.sync_copy(acc_sh.at[pl.ds(r0 + t * 160, 160)], buf_v)
        pltpu.sync_copy(buf_v, out_hbm.at[c, pl.ds(r0 + t * 160, 160)])
        return carry
    lax.fori_loop(0, _RPT // 160, dump, 0)


def _sc_degree(dst2d):
    mesh = plsc.VectorSubcoreMesh(core_axis_name="c", subcore_axis_name="s")
    f = pl.kernel(
        _deg_body,
        mesh=mesh,
        out_type=jax.ShapeDtypeStruct((2, _NP, _H), jnp.float32),
        scratch_types=[
            pltpu.VMEM((_EROWS // 32, _CHUNK), jnp.int32),
            pltpu.VMEM((_CHUNK, _H), jnp.float32),
            pltpu.VMEM((160, _H), jnp.float32),
            pltpu.VMEM_SHARED((_NP, _H), jnp.float32),
        ],
    )
    return f(dst2d)


_HALF = _NP // 2
_ACCR = _HALF + 128
_RPT2 = _HALF // _TILES
_PCH = 40


def _seg_body(src_hbm, dst_hbm, hs_hbm, out_hbm, src_v, dst_v, rowa_v, rowb_v,
              rowc_v, rowd_v, acc_sh, sema, semb, semc, semd):
    c = lax.axis_index("c")
    s = lax.axis_index("s")
    r0 = s * _RPT2
    buf = rowa_v.at[pl.ds(0, 64)]
    bufs = [rowa_v, rowb_v, rowc_v, rowd_v]
    sems = [sema, semb, semc, semd]

    def init(t, carry):
        pltpu.sync_copy(hs_hbm.at[pl.ds(c * _HALF + r0 + t * 64, 64)], buf)
        pltpu.sync_copy(buf, acc_sh.at[pl.ds(r0 + t * 64, 64)])
        return carry
    lax.fori_loop(0, _RPT2 // 64, init, 0)

    plsc.subcore_barrier()

    def phase(p, carry):
        pltpu.sync_copy(src_hbm.at[pl.ds(s * _ECHUNKS + p * _PCH, _PCH)], src_v)
        pltpu.sync_copy(
            dst_hbm.at[pl.ds((c * _TILES + s) * _ECHUNKS + p * _PCH, _PCH)], dst_v)
        for b in range(4):
            pltpu.async_copy(hs_hbm.at[src_v.at[b]], bufs[b], sems[b])

        def ring(t, carry2):
            j0 = t * 4
            for b in range(4):
                jb = j0 + b
                pltpu.make_async_copy(
                    hs_hbm.at[pl.ds(0, _CHUNK)], bufs[b], sems[b]).wait()
                pltpu.sync_copy(bufs[b], acc_sh.at[dst_v.at[jb]], add=True)

                @pl.when(jb + 4 < _PCH)
                def _(b=b, jb=jb):
                    pltpu.async_copy(hs_hbm.at[src_v.at[jb + 4]], bufs[b], sems[b])
            return carry2
        lax.fori_loop(0, _PCH // 4, ring, 0)
        return carry
    lax.fori_loop(0, _ECHUNKS // _PCH, phase, 0)

    plsc.subcore_barrier()

    def dump(t, carry):
        pltpu.sync_copy(acc_sh.at[pl.ds(r0 + t * 64, 64)], buf)
        pltpu.sync_copy(buf, out_hbm.at[c, pl.ds(r0 + t * 64, 64)])
        return carry
    lax.fori_loop(0, _RPT2 // 64, dump, 0)


def _sc_segsum(src2d, dstc, hs):
    mesh = plsc.VectorSubcoreMesh(core_axis_name="c", subcore_axis_name="s")
    f = pl.kernel(
        _seg_body,
        mesh=mesh,
        out_type=jax.ShapeDtypeStruct((2, _HALF, _H), jnp.float32),
        scratch_types=[
            pltpu.VMEM((_PCH, _CHUNK), jnp.int32),
            pltpu.VMEM((_PCH, _CHUNK), jnp.int32),
            pltpu.VMEM((_CHUNK, _H), jnp.float32),
            pltpu.VMEM((_CHUNK, _H), jnp.float32),
            pltpu.VMEM((_CHUNK, _H), jnp.float32),
            pltpu.VMEM((_CHUNK, _H), jnp.float32),
            pltpu.VMEM_SHARED((_ACCR, _H), jnp.float32),
            pltpu.SemaphoreType.DMA,
            pltpu.SemaphoreType.DMA,
            pltpu.SemaphoreType.DMA,
            pltpu.SemaphoreType.DMA,
        ],
    )
    return f(src2d, dstc, hs)


def _mm_scale_body(x_ref, w_ref, degp_ref, out_ref):
    dinv = lax.rsqrt(degp_ref[0, :, 0] + degp_ref[1, :, 0] + 1.0)
    h = jnp.dot(x_ref[...], w_ref[...], preferred_element_type=jnp.float32)
    out_ref[...] = h * dinv[:, None]


def _tc_matmul_scale(x_pad, w, degp):
    return pl.pallas_call(
        _mm_scale_body,
        grid=(_NBLK,),
        in_specs=[
            pl.BlockSpec((_BLK, _D), lambda i: (i, 0)),
            pl.BlockSpec((_D, _H), lambda i: (0, 0)),
            pl.BlockSpec((2, _BLK, _H), lambda i: (0, i, 0)),
        ],
        out_specs=pl.BlockSpec((_BLK, _H), lambda i: (i, 0)),
        out_shape=jax.ShapeDtypeStruct((_NP, _H), jnp.float32),
    )(x_pad, w, degp)


def _combine_body(agg_ref, b_ref, degp_ref, z_ref, st_ref, acc):
    i = pl.program_id(0)
    dinv = lax.rsqrt(degp_ref[0, :, 0] + degp_ref[1, :, 0] + 1.0)
    z = agg_ref[0]
    z = z * dinv[:, None] + b_ref[...]
    z_ref[...] = z
    rows = i * _BLK + lax.broadcasted_iota(jnp.int32, (_BLK, 1), 0)
    zm = jnp.where(rows < _N, z, 0.0)
    part = jnp.concatenate([
        jnp.sum(zm, axis=0, keepdims=True),
        jnp.sum(zm * zm, axis=0, keepdims=True),
    ], axis=0)

    @pl.when(i == 0)
    def _():
        acc[...] = jnp.zeros_like(acc)

    acc[...] += part
    st_ref[...] = acc[...]


def _tc_combine(agg, b_row, degp):
    return pl.pallas_call(
        _combine_body,
        grid=(_NBLK,),
        in_specs=[
            pl.BlockSpec((1, _BLK, _H), lambda i: (i // 5, i % 5, 0)),
            pl.BlockSpec((1, _H), lambda i: (0, 0)),
            pl.BlockSpec((2, _BLK, _H), lambda i: (0, i, 0)),
        ],
        out_specs=[
            pl.BlockSpec((_BLK, _H), lambda i: (i, 0)),
            pl.BlockSpec((2, _H), lambda i: (0, 0)),
        ],
        out_shape=[
            jax.ShapeDtypeStruct((_NP, _H), jnp.float32),
            jax.ShapeDtypeStruct((2, _H), jnp.float32),
        ],
        scratch_shapes=[pltpu.VMEM((2, _H), jnp.float32)],
    )(agg, b_row, degp)


def _norm_mm_body(z_ref, st_ref, w_ref, degp_ref, out_ref):
    mean = st_ref[0, :] / float(_N)
    var = st_ref[1, :] / float(_N) - mean * mean
    inv = lax.rsqrt(var + 1e-5)
    hn = jnp.maximum((z_ref[...] - mean[None, :]) * inv[None, :], 0.0)
    h = jnp.dot(hn, w_ref[...], preferred_element_type=jnp.float32)
    dinv = lax.rsqrt(degp_ref[0, :, 0] + degp_ref[1, :, 0] + 1.0)
    out_ref[...] = h * dinv[:, None]


def _tc_norm_matmul(z, st, w, degp):
    return pl.pallas_call(
        _norm_mm_body,
        grid=(_NBLK,),
        in_specs=[
            pl.BlockSpec((_BLK, _H), lambda i: (i, 0)),
            pl.BlockSpec((2, _H), lambda i: (0, 0)),
            pl.BlockSpec((_H, _H), lambda i: (0, 0)),
            pl.BlockSpec((2, _BLK, _H), lambda i: (0, i, 0)),
        ],
        out_specs=pl.BlockSpec((_BLK, _H), lambda i: (i, 0)),
        out_shape=jax.ShapeDtypeStruct((_NP, _H), jnp.float32),
    )(z, st, w, degp)


def _final_body(cur_ref, z_ref, zc_ref, st_ref, part_ref, h_ref, p_ref, acc):
    i = pl.program_id(0)
    mean = st_ref[0, :] / float(_N)
    var = st_ref[1, :] / float(_N) - mean * mean
    inv = lax.rsqrt(var + 1e-5)
    hb = jnp.maximum((z_ref[...] - mean[None, :]) * inv[None, :], 0.0)
    h_ref[...] = hb
    sub = cur_ref[0] % 8
    lane = lax.broadcasted_iota(jnp.int32, (8, 1), 0)
    zc = jnp.sum(jnp.where(lane == sub, zc_ref[...], 0.0), axis=0, keepdims=True)
    hc = jnp.maximum((zc - mean[None, :]) * inv[None, :], 0.0)
    scores = jnp.sum(hb * hc, axis=1, keepdims=True)
    rows = i * _BLK + lax.broadcasted_iota(jnp.int32, (_BLK, 1), 0)
    scores = jnp.where(rows < _N, scores, 0.0)
    contrib = jnp.sum(part_ref[...] * scores, axis=0, keepdims=True)

    @pl.when(i == 0)
    def _():
        acc[...] = jnp.zeros_like(acc)

    acc[...] += contrib
    p_ref[...] = acc[...]


def _tc_final(cur, z, st, part_pad):
    grid_spec = pltpu.PrefetchScalarGridSpec(
        num_scalar_prefetch=1,
        grid=(_NBLK,),
        in_specs=[
            pl.BlockSpec((_BLK, _H), lambda i, cur: (i, 0)),
            pl.BlockSpec((8, _H), lambda i, cur: (cur[0] // 8, 0)),
            pl.BlockSpec((2, _H), lambda i, cur: (0, 0)),
            pl.BlockSpec((_BLK, _H), lambda i, cur: (i, 0)),
        ],
        out_specs=[
            pl.BlockSpec((_BLK, _H), lambda i, cur: (i, 0)),
            pl.BlockSpec((1, _H), lambda i, cur: (0, 0)),
        ],
        scratch_shapes=[pltpu.VMEM((1, _H), jnp.float32)],
    )
    return pl.pallas_call(
        _final_body,
        grid_spec=grid_spec,
        out_shape=[
            jax.ShapeDtypeStruct((_N, _H), jnp.float32),
            jax.ShapeDtypeStruct((1, _H), jnp.float32),
        ],
    )(cur, z, z, st, part_pad)


def kernel(x, edge_index, curr_node_id, partitions, node_weights, W1, b1, W2, b2):
    del node_weights
    e = edge_index.shape[1]
    pad = jnp.full((_EP - e,), _N, dtype=jnp.int32)
    src_p = jnp.concatenate([edge_index[0].astype(jnp.int32), pad])
    dst_p = jnp.concatenate([edge_index[1].astype(jnp.int32), pad])
    src2d = src_p.reshape(_EROWS, _CHUNK)
    dst2d = dst_p.reshape(_EROWS, _CHUNK)
    sent = _HALF + (jnp.arange(_EP, dtype=jnp.int32) % 128)
    dst_c0 = jnp.where(dst_p < _HALF, dst_p, sent)
    dst_c1 = jnp.where(dst_p >= _HALF, dst_p - _HALF, sent)
    dstc = jnp.concatenate([dst_c0, dst_c1]).reshape(2 * _EROWS, _CHUNK)
    x_pad = jnp.pad(x, ((0, _NP - _N), (0, 0)))
    part_pad = jnp.pad(partitions, ((0, _NP - _N), (0, _H - _P)))

    degp = _sc_degree(dst2d)
    hs1 = _tc_matmul_scale(x_pad, W1, degp)
    agg1 = _sc_segsum(src2d, dstc, hs1)
    z1, st1 = _tc_combine(agg1, b1.reshape(1, _H), degp)
    hs2 = _tc_norm_matmul(z1, st1, W2, degp)
    agg2 = _sc_segsum(src2d, dstc, hs2)
    z2, st2 = _tc_combine(agg2, b2.reshape(1, _H), degp)

    cur = jnp.asarray(curr_node_id, dtype=jnp.int32).reshape(1)
    h, p = _tc_final(cur, z2, st2, part_pad)
    return p[:, :_P], h


# speedup vs baseline: 15.7860x; 1.0044x over previous
import functools

import jax
import jax.numpy as jnp
from jax import lax
from jax.experimental import pallas as pl
from jax.experimental.pallas import tpu as pltpu
from jax.experimental.pallas import tpu_sc as plsc

_N = 10000
_E = 320000
_D = 128
_H = 128
_P = 100

_NP = 10240
_EP = 327680
_EROWS = 2560
_CHUNK = 128
_TILES = 16
_RPT = _NP // _TILES
_ECHUNKS = _EROWS // _TILES
_BLK = 1024
_NBLK = _NP // _BLK


def _deg_body(dst_hbm, out_hbm, dst_v, ones_v, buf_v, acc_sh):
    c = lax.axis_index("c")
    s = lax.axis_index("s")
    w = c * _TILES + s
    rows = _EROWS // 32
    r0 = s * _RPT

    def fill_zero(i, carry):
        j = i // 8
        k = i % 8
        buf_v[j, pl.ds(k * 16, 16)] = jnp.zeros((16,), jnp.float32)
        return carry
    lax.fori_loop(0, 160 * 8, fill_zero, 0)

    def fill_one(i, carry):
        j = i // 8
        k = i % 8
        ones_v[j, pl.ds(k * 16, 16)] = jnp.ones((16,), jnp.float32)
        return carry
    lax.fori_loop(0, _CHUNK * 8, fill_one, 0)

    def init(t, carry):
        pltpu.sync_copy(buf_v, acc_sh.at[pl.ds(r0 + t * 160, 160)])
        return carry
    lax.fori_loop(0, _RPT // 160, init, 0)

    pltpu.sync_copy(dst_hbm.at[pl.ds(w * rows, rows)], dst_v)
    plsc.subcore_barrier()

    def upd(j, carry):
        pltpu.sync_copy(ones_v, acc_sh.at[dst_v.at[j]], add=True)
        return carry
    lax.fori_loop(0, rows, upd, 0)

    plsc.subcore_barrier()

    def dump(t, carry):
        pltpu.sync_copy(acc_sh.at[pl.ds(r0 + t * 160, 160)], buf_v)
        pltpu.sync_copy(buf_v, out_hbm.at[c, pl.ds(r0 + t * 160, 160)])
        return carry
    lax.fori_loop(0, _RPT // 160, dump, 0)


def _sc_degree(dst2d):
    mesh = plsc.VectorSubcoreMesh(core_axis_name="c", subcore_axis_name="s")
    f = pl.kernel(
        _deg_body,
        mesh=mesh,
        out_type=jax.ShapeDtypeStruct((2, _NP, _H), jnp.float32),
        scratch_types=[
            pltpu.VMEM((_EROWS // 32, _CHUNK), jnp.int32),
            pltpu.VMEM((_CHUNK, _H), jnp.float32),
            pltpu.VMEM((160, _H), jnp.float32),
            pltpu.VMEM_SHARED((_NP, _H), jnp.float32),
        ],
    )
    return f(dst2d)


_HALF = _NP // 2
_ACCR = _HALF + 128
_RPT2 = _HALF // _TILES
_PCH = 40


def _seg_body(src_hbm, dst_hbm, hs_hbm, out_hbm, src_v, dst_v, rowa_v, rowb_v,
              rowc_v, rowd_v, acc_sh, sema, semb, semc, semd, ssa, ssb, ssc, ssd):
    c = lax.axis_index("c")
    s = lax.axis_index("s")
    r0 = s * _RPT2
    buf = rowa_v.at[pl.ds(0, 64)]
    bufs = [rowa_v, rowb_v, rowc_v, rowd_v]
    sems = [sema, semb, semc, semd]
    ssems = [ssa, ssb, ssc, ssd]

    def drain(b):
        pltpu.make_async_copy(hs_hbm.at[pl.ds(0, _CHUNK)], bufs[b], ssems[b]).wait()

    def init(t, carry):
        pltpu.sync_copy(hs_hbm.at[pl.ds(c * _HALF + r0 + t * 64, 64)], buf)
        pltpu.sync_copy(buf, acc_sh.at[pl.ds(r0 + t * 64, 64)])
        return carry
    lax.fori_loop(0, _RPT2 // 64, init, 0)

    plsc.subcore_barrier()

    def phase(p, carry):
        pltpu.sync_copy(src_hbm.at[pl.ds(s * _ECHUNKS + p * _PCH, _PCH)], src_v)
        pltpu.sync_copy(
            dst_hbm.at[pl.ds((c * _TILES + s) * _ECHUNKS + p * _PCH, _PCH)], dst_v)

        @pl.when(p > 0)
        def _():
            for b in range(4):
                drain(b)

        for b in range(4):
            pltpu.async_copy(hs_hbm.at[src_v.at[b]], bufs[b], sems[b])

        def ring(t, carry2):
            j0 = t * 4
            for b in range(4):
                jb = j0 + b
                pltpu.make_async_copy(
                    hs_hbm.at[pl.ds(0, _CHUNK)], bufs[b], sems[b]).wait()
                pltpu.async_copy(bufs[b], acc_sh.at[dst_v.at[jb]], ssems[b],
                                 add=True)

                @pl.when(jb + 4 < _PCH)
                def _(b=b, jb=jb):
                    drain(b)
                    pltpu.async_copy(hs_hbm.at[src_v.at[jb + 4]], bufs[b], sems[b])
            return carry2
        lax.fori_loop(0, _PCH // 4, ring, 0)
        return carry
    lax.fori_loop(0, _ECHUNKS // _PCH, phase, 0)

    for b in range(4):
        drain(b)

    plsc.subcore_barrier()

    def dump(t, carry):
        pltpu.sync_copy(acc_sh.at[pl.ds(r0 + t * 64, 64)], buf)
        pltpu.sync_copy(buf, out_hbm.at[c, pl.ds(r0 + t * 64, 64)])
        return carry
    lax.fori_loop(0, _RPT2 // 64, dump, 0)


def _sc_segsum(src2d, dstc, hs):
    mesh = plsc.VectorSubcoreMesh(core_axis_name="c", subcore_axis_name="s")
    f = pl.kernel(
        _seg_body,
        mesh=mesh,
        out_type=jax.ShapeDtypeStruct((2, _HALF, _H), jnp.float32),
        scratch_types=[
            pltpu.VMEM((_PCH, _CHUNK), jnp.int32),
            pltpu.VMEM((_PCH, _CHUNK), jnp.int32),
            pltpu.VMEM((_CHUNK, _H), jnp.float32),
            pltpu.VMEM((_CHUNK, _H), jnp.float32),
            pltpu.VMEM((_CHUNK, _H), jnp.float32),
            pltpu.VMEM((_CHUNK, _H), jnp.float32),
            pltpu.VMEM_SHARED((_ACCR, _H), jnp.float32),
            pltpu.SemaphoreType.DMA,
            pltpu.SemaphoreType.DMA,
            pltpu.SemaphoreType.DMA,
            pltpu.SemaphoreType.DMA,
            pltpu.SemaphoreType.DMA,
            pltpu.SemaphoreType.DMA,
            pltpu.SemaphoreType.DMA,
            pltpu.SemaphoreType.DMA,
        ],
    )
    return f(src2d, dstc, hs)


def _mm_scale_body(x_ref, w_ref, degp_ref, out_ref):
    dinv = lax.rsqrt(degp_ref[0, :, 0] + degp_ref[1, :, 0] + 1.0)
    h = jnp.dot(x_ref[...], w_ref[...], preferred_element_type=jnp.float32)
    out_ref[...] = h * dinv[:, None]


def _tc_matmul_scale(x_pad, w, degp):
    return pl.pallas_call(
        _mm_scale_body,
        grid=(_NBLK,),
        in_specs=[
            pl.BlockSpec((_BLK, _D), lambda i: (i, 0)),
            pl.BlockSpec((_D, _H), lambda i: (0, 0)),
            pl.BlockSpec((2, _BLK, _H), lambda i: (0, i, 0)),
        ],
        out_specs=pl.BlockSpec((_BLK, _H), lambda i: (i, 0)),
        out_shape=jax.ShapeDtypeStruct((_NP, _H), jnp.float32),
    )(x_pad, w, degp)


def _combine_body(agg_ref, b_ref, degp_ref, z_ref, st_ref, acc):
    i = pl.program_id(0)
    dinv = lax.rsqrt(degp_ref[0, :, 0] + degp_ref[1, :, 0] + 1.0)
    z = agg_ref[0]
    z = z * dinv[:, None] + b_ref[...]
    z_ref[...] = z
    rows = i * _BLK + lax.broadcasted_iota(jnp.int32, (_BLK, 1), 0)
    zm = jnp.where(rows < _N, z, 0.0)
    part = jnp.concatenate([
        jnp.sum(zm, axis=0, keepdims=True),
        jnp.sum(zm * zm, axis=0, keepdims=True),
    ], axis=0)

    @pl.when(i == 0)
    def _():
        acc[...] = jnp.zeros_like(acc)

    acc[...] += part
    st_ref[...] = acc[...]


def _tc_combine(agg, b_row, degp):
    return pl.pallas_call(
        _combine_body,
        grid=(_NBLK,),
        in_specs=[
            pl.BlockSpec((1, _BLK, _H), lambda i: (i // 5, i % 5, 0)),
            pl.BlockSpec((1, _H), lambda i: (0, 0)),
            pl.BlockSpec((2, _BLK, _H), lambda i: (0, i, 0)),
        ],
        out_specs=[
            pl.BlockSpec((_BLK, _H), lambda i: (i, 0)),
            pl.BlockSpec((2, _H), lambda i: (0, 0)),
        ],
        out_shape=[
            jax.ShapeDtypeStruct((_NP, _H), jnp.float32),
            jax.ShapeDtypeStruct((2, _H), jnp.float32),
        ],
        scratch_shapes=[pltpu.VMEM((2, _H), jnp.float32)],
    )(agg, b_row, degp)


def _norm_mm_body(z_ref, st_ref, w_ref, degp_ref, out_ref):
    mean = st_ref[0, :] / float(_N)
    var = st_ref[1, :] / float(_N) - mean * mean
    inv = lax.rsqrt(var + 1e-5)
    hn = jnp.maximum((z_ref[...] - mean[None, :]) * inv[None, :], 0.0)
    h = jnp.dot(hn, w_ref[...], preferred_element_type=jnp.float32)
    dinv = lax.rsqrt(degp_ref[0, :, 0] + degp_ref[1, :, 0] + 1.0)
    out_ref[...] = h * dinv[:, None]


def _tc_norm_matmul(z, st, w, degp):
    return pl.pallas_call(
        _norm_mm_body,
        grid=(_NBLK,),
        in_specs=[
            pl.BlockSpec((_BLK, _H), lambda i: (i, 0)),
            pl.BlockSpec((2, _H), lambda i: (0, 0)),
            pl.BlockSpec((_H, _H), lambda i: (0, 0)),
            pl.BlockSpec((2, _BLK, _H), lambda i: (0, i, 0)),
        ],
        out_specs=pl.BlockSpec((_BLK, _H), lambda i: (i, 0)),
        out_shape=jax.ShapeDtypeStruct((_NP, _H), jnp.float32),
    )(z, st, w, degp)


def _final_body(cur_ref, z_ref, zc_ref, st_ref, part_ref, h_ref, p_ref, acc):
    i = pl.program_id(0)
    mean = st_ref[0, :] / float(_N)
    var = st_ref[1, :] / float(_N) - mean * mean
    inv = lax.rsqrt(var + 1e-5)
    hb = jnp.maximum((z_ref[...] - mean[None, :]) * inv[None, :], 0.0)
    h_ref[...] = hb
    sub = cur_ref[0] % 8
    lane = lax.broadcasted_iota(jnp.int32, (8, 1), 0)
    zc = jnp.sum(jnp.where(lane == sub, zc_ref[...], 0.0), axis=0, keepdims=True)
    hc = jnp.maximum((zc - mean[None, :]) * inv[None, :], 0.0)
    scores = jnp.sum(hb * hc, axis=1, keepdims=True)
    rows = i * _BLK + lax.broadcasted_iota(jnp.int32, (_BLK, 1), 0)
    scores = jnp.where(rows < _N, scores, 0.0)
    contrib = jnp.sum(part_ref[...] * scores, axis=0, keepdims=True)

    @pl.when(i == 0)
    def _():
        acc[...] = jnp.zeros_like(acc)

    acc[...] += contrib
    p_ref[...] = acc[...]


def _tc_final(cur, z, st, part_pad):
    grid_spec = pltpu.PrefetchScalarGridSpec(
        num_scalar_prefetch=1,
        grid=(_NBLK,),
        in_specs=[
            pl.BlockSpec((_BLK, _H), lambda i, cur: (i, 0)),
            pl.BlockSpec((8, _H), lambda i, cur: (cur[0] // 8, 0)),
            pl.BlockSpec((2, _H), lambda i, cur: (0, 0)),
            pl.BlockSpec((_BLK, _H), lambda i, cur: (i, 0)),
        ],
        out_specs=[
            pl.BlockSpec((_BLK, _H), lambda i, cur: (i, 0)),
            pl.BlockSpec((1, _H), lambda i, cur: (0, 0)),
        ],
        scratch_shapes=[pltpu.VMEM((1, _H), jnp.float32)],
    )
    return pl.pallas_call(
        _final_body,
        grid_spec=grid_spec,
        out_shape=[
            jax.ShapeDtypeStruct((_N, _H), jnp.float32),
            jax.ShapeDtypeStruct((1, _H), jnp.float32),
        ],
    )(cur, z, z, st, part_pad)


def kernel(x, edge_index, curr_node_id, partitions, node_weights, W1, b1, W2, b2):
    del node_weights
    e = edge_index.shape[1]
    pad = jnp.full((_EP - e,), _N, dtype=jnp.int32)
    src_p = jnp.concatenate([edge_index[0].astype(jnp.int32), pad])
    dst_p = jnp.concatenate([edge_index[1].astype(jnp.int32), pad])
    src2d = src_p.reshape(_EROWS, _CHUNK)
    dst2d = dst_p.reshape(_EROWS, _CHUNK)
    sent = _HALF + (jnp.arange(_EP, dtype=jnp.int32) % 128)
    dst_c0 = jnp.where(dst_p < _HALF, dst_p, sent)
    dst_c1 = jnp.where(dst_p >= _HALF, dst_p - _HALF, sent)
    dstc = jnp.concatenate([dst_c0, dst_c1]).reshape(2 * _EROWS, _CHUNK)
    x_pad = jnp.pad(x, ((0, _NP - _N), (0, 0)))
    part_pad = jnp.pad(partitions, ((0, _NP - _N), (0, _H - _P)))

    degp = _sc_degree(dst2d)
    hs1 = _tc_matmul_scale(x_pad, W1, degp)
    agg1 = _sc_segsum(src2d, dstc, hs1)
    z1, st1 = _tc_combine(agg1, b1.reshape(1, _H), degp)
    hs2 = _tc_norm_matmul(z1, st1, W2, degp)
    agg2 = _sc_segsum(src2d, dstc, hs2)
    z2, st2 = _tc_combine(agg2, b2.reshape(1, _H), degp)

    cur = jnp.asarray(curr_node_id, dtype=jnp.int32).reshape(1)
    h, p = _tc_final(cur, z2, st2, part_pad)
    return p[:, :_P], h
